# baseline (speedup 1.0000x reference)
import jax
import jax.numpy as jnp
from jax.experimental import pallas as pl
from jax.experimental.pallas import tpu as pltpu

_B = 4096
_E = 1000
_MARGIN = 0.9

_WIDTHS = (1024, 1024, 1024, 512, 256, 128, 128)
_OFFS = (0, 1024, 2048, 3072, 3584, 3840, 3968)


def _loss_body(out_hbm, tgt_ref, acc_ref, b0, b1, b2, sems, vacc_ref):
    bufs = (b0, b1, b2)

    def cp(k):
        w = _WIDTHS[k]
        return pltpu.make_async_copy(
            out_hbm.at[:, pl.ds(_OFFS[k], w)],
            bufs[k % 3].at[:, pl.ds(0, w)],
            sems.at[k % 3],
        )

    cp(0).start()
    cp(1).start()
    vacc_ref[...] = jnp.zeros((8, 1024), jnp.float32)
    for k, w in enumerate(_WIDTHS):
        cp(k).wait()
        if k + 2 < len(_WIDTHS):
            cp(k + 2).start()
        out = bufs[k % 3][:, :w]
        tgt = tgt_ref[pl.ds(_OFFS[k], w)].reshape(1, w)
        cls = jax.lax.broadcasted_iota(jnp.int32, (_E, w), 0)
        at = jnp.sum(jnp.where(cls == tgt, out, 0.0), axis=0, keepdims=True)
        d = jnp.maximum((_MARGIN - at) + out, 0.0)
        vacc_ref[:, :w] += jnp.sum((d * d).reshape(_E // 8, 8, w), axis=0)
    acc_ref[...] = jnp.full((1, 1), jnp.sum(vacc_ref[...]) * (1.0 / _B),
                            jnp.float32)


def kernel(output, target):
    out_t = output.T
    acc = pl.pallas_call(
        _loss_body,
        in_specs=[
            pl.BlockSpec(memory_space=pltpu.MemorySpace.HBM),
            pl.BlockSpec(memory_space=pltpu.VMEM),
        ],
        out_specs=pl.BlockSpec(memory_space=pltpu.VMEM),
        out_shape=jax.ShapeDtypeStruct((1, 1), jnp.float32),
        scratch_shapes=[
            pltpu.VMEM((_E, 1024), jnp.float32),
            pltpu.VMEM((_E, 1024), jnp.float32),
            pltpu.VMEM((_E, 1024), jnp.float32),
            pltpu.SemaphoreType.DMA((3,)),
            pltpu.VMEM((8, 1024), jnp.float32),
        ],
    )(out_t, target.astype(jnp.int32))
    return acc[0, 0]


# speedup vs baseline: 1.0883x; 1.0883x over previous
import jax
import jax.numpy as jnp
from jax.experimental import pallas as pl
from jax.experimental.pallas import tpu as pltpu

_B = 4096
_E = 1000
_MARGIN = 0.9

_WIDTHS = (1024, 1024, 1024, 512, 512)
_OFFS = (0, 1024, 2048, 3072, 3584)


def _loss_body(out_hbm, tgt_ref, acc_ref, b0, b1, b2, sems, vacc_ref):
    bufs = (b0, b1, b2)

    def cp(k):
        w = _WIDTHS[k]
        return pltpu.make_async_copy(
            out_hbm.at[:, pl.ds(_OFFS[k], w)],
            bufs[k % 3].at[:, pl.ds(0, w)],
            sems.at[k % 3],
        )

    cp(0).start()
    cp(1).start()
    vacc_ref[...] = jnp.zeros((8, 1024), jnp.float32)
    for k, w in enumerate(_WIDTHS):
        cp(k).wait()
        if k + 2 < len(_WIDTHS):
            cp(k + 2).start()
        out = bufs[k % 3][:, :w]
        tgt = tgt_ref[pl.ds(_OFFS[k], w)].reshape(1, w)
        cls = jax.lax.broadcasted_iota(jnp.int32, (_E, w), 0)
        at = jnp.sum(jnp.where(cls == tgt, out, 0.0), axis=0, keepdims=True)
        d = jnp.maximum((_MARGIN - at) + out, 0.0)
        vacc_ref[:, :w] += jnp.sum((d * d).reshape(_E // 8, 8, w), axis=0)
    acc_ref[...] = jnp.full((1, 1), jnp.sum(vacc_ref[...]) * (1.0 / _B),
                            jnp.float32)


def kernel(output, target):
    out_t = output.T
    acc = pl.pallas_call(
        _loss_body,
        in_specs=[
            pl.BlockSpec(memory_space=pltpu.MemorySpace.HBM),
            pl.BlockSpec(memory_space=pltpu.VMEM),
        ],
        out_specs=pl.BlockSpec(memory_space=pltpu.VMEM),
        out_shape=jax.ShapeDtypeStruct((1, 1), jnp.float32),
        scratch_shapes=[
            pltpu.VMEM((_E, 1024), jnp.float32),
            pltpu.VMEM((_E, 1024), jnp.float32),
            pltpu.VMEM((_E, 1024), jnp.float32),
            pltpu.SemaphoreType.DMA((3,)),
            pltpu.VMEM((8, 1024), jnp.float32),
        ],
    )(out_t, target.astype(jnp.int32))
    return acc[0, 0]


# speedup vs baseline: 1.1734x; 1.0782x over previous
import jax
import jax.numpy as jnp
from jax.experimental import pallas as pl
from jax.experimental.pallas import tpu as pltpu

_B = 4096
_E = 1000
_BL = 1024
_MARGIN = 0.9


def _loss_body(out_ref, tgt_ref, acc_ref, vacc_ref):
    i = pl.program_id(0)

    @pl.when(i == 0)
    def _():
        vacc_ref[...] = jnp.zeros((8, _BL), jnp.float32)

    out = out_ref[...]
    tgt = tgt_ref[...].reshape(1, _BL)
    cls = jax.lax.broadcasted_iota(jnp.int32, (_E, _BL), 0)
    at = jnp.sum(jnp.where(cls == tgt, out, 0.0), axis=0, keepdims=True)
    d = jnp.maximum((_MARGIN - at) + out, 0.0)
    vacc_ref[...] += jnp.sum((d * d).reshape(_E // 8, 8, _BL), axis=0)

    @pl.when(i == pl.num_programs(0) - 1)
    def _():
        acc_ref[...] = jnp.full((1, 1), jnp.sum(vacc_ref[...]) * (1.0 / _B),
                                jnp.float32)


def kernel(output, target):
    out_t = output.T
    acc = pl.pallas_call(
        _loss_body,
        grid=(_B // _BL,),
        in_specs=[
            pl.BlockSpec((_E, _BL), lambda i: (0, i)),
            pl.BlockSpec((_BL,), lambda i: (i,)),
        ],
        out_specs=pl.BlockSpec((1, 1), lambda i: (0, 0)),
        out_shape=jax.ShapeDtypeStruct((1, 1), jnp.float32),
        scratch_shapes=[pltpu.VMEM((8, _BL), jnp.float32)],
    )(out_t, target.astype(jnp.int32))
    return acc[0, 0]
